# baseline (device time: 159499 ns/iter reference)
import jax
import jax.numpy as jnp
from jax import lax
from jax.experimental import pallas as pl
from jax.experimental.pallas import tpu as pltpu

T_PER = 1024
D = 1024
F = 4096
E_LOCAL = 8
N_E = 2 * E_LOCAL
K = 2
C = 320
N_SLOTS = E_LOCAL * C
FB = 512
NF = F // FB

N_FP = 4
NF_PER = NF // N_FP

N_PIPE = 8
SEG = T_PER // N_PIPE


def _peer():
    return (lax.axis_index("x"), 1 - lax.axis_index("y"), lax.axis_index("z"))


def _neighbor_barrier(peer):
    barrier_sem = pltpu.get_barrier_semaphore()
    pl.semaphore_signal(
        barrier_sem, inc=1, device_id=peer, device_id_type=pl.DeviceIdType.MESH
    )
    pl.semaphore_wait(barrier_sem, 1)


def _gate_exch_body(
    x_ref, rt_ref, xfull_ref, idxfull_ref, wfull_ref, rfull_ref,
    send_sems, recv_sems
):
    my_y = lax.axis_index("y")
    peer = _peer()
    _neighbor_barrier(peer)

    def exch(i, ref):
        return pltpu.make_async_remote_copy(
            src_ref=ref.at[my_y],
            dst_ref=ref.at[my_y],
            send_sem=send_sems.at[i],
            recv_sem=recv_sems.at[i],
            device_id=peer,
            device_id_type=pl.DeviceIdType.MESH,
        )

    xfull_ref[pl.ds(my_y, 1)] = x_ref[...].astype(jnp.bfloat16)[None]
    rd_x = exch(0, xfull_ref)
    rd_x.start()

    rfull_ref[pl.ds(my_y, 1)] = rt_ref[...][None]
    rd_r = exch(1, rfull_ref)
    rd_r.start()
    rd_r.wait()

    rcat = jnp.concatenate([rfull_ref[0], rfull_ref[1]], axis=0)
    g = lax.dot_general(
        x_ref[...], rcat, (((1,), (1,)), ((), ())),
        preferred_element_type=jnp.float32,
        precision=lax.Precision.HIGHEST,
    )
    iota16 = lax.broadcasted_iota(jnp.int32, (T_PER, N_E), 1)
    m1 = jnp.max(g, axis=1)
    i1 = jnp.argmax(g, axis=1).astype(jnp.int32)
    g2 = jnp.where(iota16 == i1[:, None], -jnp.inf, g)
    m2 = jnp.max(g2, axis=1)
    i2 = jnp.argmax(g2, axis=1).astype(jnp.int32)
    w1 = 1.0 / (1.0 + jnp.exp(m2 - m1))

    idxfull_ref[pl.ds(my_y, 1)] = jnp.concatenate(
        [i1[None, None, :], i2[None, None, :]], axis=1
    )
    wfull_ref[pl.ds(my_y, 1)] = jnp.concatenate(
        [w1[None, None, :], (1.0 - w1)[None, None, :]], axis=1
    )
    rd_i = exch(2, idxfull_ref)
    rd_w = exch(3, wfull_ref)
    rd_i.start()
    rd_w.start()
    rd_i.wait()
    rd_w.wait()
    rd_x.wait()


def _gate_exch(x, rt):
    return pl.pallas_call(
        _gate_exch_body,
        out_shape=(
            jax.ShapeDtypeStruct((2, T_PER, D), jnp.bfloat16),
            jax.ShapeDtypeStruct((2, K, T_PER), jnp.int32),
            jax.ShapeDtypeStruct((2, K, T_PER), jnp.float32),
        ),
        in_specs=[pl.BlockSpec(memory_space=pltpu.VMEM)] * 2,
        out_specs=(pl.BlockSpec(memory_space=pltpu.VMEM),) * 3,
        scratch_shapes=[
            pltpu.VMEM((2, E_LOCAL, D), jnp.float32),
            pltpu.SemaphoreType.DMA((4,)),
            pltpu.SemaphoreType.DMA((4,)),
        ],
        compiler_params=pltpu.CompilerParams(collective_id=0),
    )(x, rt)


def _moe_body(
    fp_ref, xf_ref, s2_ref, w2a_ref, w1_ref, w2_ref, out_ref,
    xg_ref, acc_ref, wg_ref, part_ref,
    send_buf, rbuf_y, rbuf_x, rbuf_z, send_sems, recv_sems
):
    e = pl.program_id(0)
    f = pl.program_id(1)

    @pl.when(f == 0)
    def _():
        iota_c = lax.broadcasted_iota(jnp.int32, (C, 2 * T_PER), 0) + e * C
        oh0 = iota_c == s2_ref[0][None, :]
        oh1 = iota_c == s2_ref[1][None, :]
        ohc = oh0.astype(jnp.bfloat16) + oh1.astype(jnp.bfloat16)
        xg_ref[...] = jnp.dot(
            ohc, xf_ref[...], preferred_element_type=jnp.float32
        ).astype(jnp.bfloat16)
        wg_ref[0, :] = jnp.sum(
            jnp.where(oh0, w2a_ref[0][None, :], 0.0)
            + jnp.where(oh1, w2a_ref[1][None, :], 0.0),
            axis=1,
        )
        acc_ref[...] = jnp.zeros_like(acc_ref)

    h = jnp.dot(
        xg_ref[...],
        w1_ref[0].astype(jnp.bfloat16),
        preferred_element_type=jnp.float32,
    )
    h = jnp.maximum(h, 0.0).astype(jnp.bfloat16)
    acc_ref[...] += jnp.dot(
        h, w2_ref[0].astype(jnp.bfloat16), preferred_element_type=jnp.float32
    )

    @pl.when(f == NF_PER - 1)
    def _():
        y_e = (acc_ref[...] * wg_ref[0, :][:, None]).astype(jnp.bfloat16)
        for half in range(2):
            rows = pl.ds(half * T_PER, T_PER)
            iota_t = lax.broadcasted_iota(jnp.int32, (T_PER, C), 1) + e * C
            st0 = s2_ref[0, rows]
            st1 = s2_ref[1, rows]
            oht = (iota_t == st0[:, None]).astype(jnp.bfloat16) + (
                iota_t == st1[:, None]
            ).astype(jnp.bfloat16)
            contrib = jnp.dot(oht, y_e, preferred_element_type=jnp.float32)

            @pl.when(e == 0)
            def _():
                part_ref[rows, :] = contrib

            @pl.when(e > 0)
            def _():
                part_ref[rows, :] += contrib

    @pl.when(jnp.logical_and(e == E_LOCAL - 1, f == NF_PER - 1))
    def _():
        my_x = lax.axis_index("x")
        my_y = lax.axis_index("y")
        my_z = lax.axis_index("z")
        n_y = (my_x, 1 - my_y, my_z)
        n_x = (1 - my_x, my_y, my_z)
        n_z = (my_x, my_y, 1 - my_z)

        barrier_sem = pltpu.get_barrier_semaphore()
        for nbr in (n_y, n_x, n_z):
            pl.semaphore_signal(
                barrier_sem, inc=1, device_id=nbr,
                device_id_type=pl.DeviceIdType.MESH,
            )
        pl.semaphore_wait(barrier_sem, 3)

        def exchange(sem_i, src, dst, q, nbr):
            return pltpu.make_async_remote_copy(
                src_ref=src.at[pl.ds(q * SEG, SEG)],
                dst_ref=dst.at[pl.ds(q * SEG, SEG)],
                send_sem=send_sems.at[sem_i],
                recv_sem=recv_sems.at[sem_i],
                device_id=nbr,
                device_id_type=pl.DeviceIdType.MESH,
            )

        peer_base = (1 - my_y) * T_PER
        my_base = my_y * T_PER
        rd_y = [None] * N_PIPE
        rd_x = [None] * N_PIPE
        rd_z = [None] * N_PIPE

        for q in range(N_PIPE):
            send_buf[pl.ds(q * SEG, SEG), :] = part_ref[
                pl.ds(peer_base + q * SEG, SEG), :
            ].astype(jnp.bfloat16)
            rd_y[q] = exchange(q, send_buf, rbuf_y, q, n_y)
            rd_y[q].start()

        for q in range(N_PIPE):
            rd_y[q].wait()
            rows_m = pl.ds(my_base + q * SEG, SEG)
            rows_s = pl.ds(q * SEG, SEG)
            part_ref[rows_m, :] += rbuf_y[rows_s, :].astype(jnp.float32)
            send_buf[rows_s, :] = part_ref[rows_m, :].astype(jnp.bfloat16)
            rd_x[q] = exchange(N_PIPE + q, send_buf, rbuf_x, q, n_x)
            rd_x[q].start()

        for q in range(N_PIPE):
            rd_x[q].wait()
            rows_m = pl.ds(my_base + q * SEG, SEG)
            rows_s = pl.ds(q * SEG, SEG)
            part_ref[rows_m, :] += rbuf_x[rows_s, :].astype(jnp.float32)
            send_buf[rows_s, :] = part_ref[rows_m, :].astype(jnp.bfloat16)
            rd_z[q] = exchange(2 * N_PIPE + q, send_buf, rbuf_z, q, n_z)
            rd_z[q].start()

        for q in range(N_PIPE):
            rd_z[q].wait()
            rows_m = pl.ds(my_base + q * SEG, SEG)
            rows_s = pl.ds(q * SEG, SEG)
            out_ref[rows_s, :] = part_ref[rows_m, :] + rbuf_z[
                rows_s, :
            ].astype(jnp.float32)


def _moe(fp, xf_bf, s2, w2, W1, W2):
    grid_spec = pltpu.PrefetchScalarGridSpec(
        num_scalar_prefetch=1,
        grid=(E_LOCAL, NF_PER),
        in_specs=[
            pl.BlockSpec((2 * T_PER, D), lambda e, f, fp: (0, 0)),
            pl.BlockSpec((K, 2 * T_PER), lambda e, f, fp: (0, 0)),
            pl.BlockSpec((K, 2 * T_PER), lambda e, f, fp: (0, 0)),
            pl.BlockSpec((1, D, FB), lambda e, f, fp: (e, 0, fp[0] * NF_PER + f)),
            pl.BlockSpec((1, FB, D), lambda e, f, fp: (e, fp[0] * NF_PER + f, 0)),
        ],
        out_specs=pl.BlockSpec((T_PER, D), lambda e, f, fp: (0, 0)),
        scratch_shapes=[
            pltpu.VMEM((C, D), jnp.bfloat16),
            pltpu.VMEM((C, D), jnp.float32),
            pltpu.VMEM((1, C), jnp.float32),
            pltpu.VMEM((2 * T_PER, D), jnp.float32),
            pltpu.VMEM((T_PER, D), jnp.bfloat16),
            pltpu.VMEM((T_PER, D), jnp.bfloat16),
            pltpu.VMEM((T_PER, D), jnp.bfloat16),
            pltpu.VMEM((T_PER, D), jnp.bfloat16),
            pltpu.SemaphoreType.DMA((3 * N_PIPE,)),
            pltpu.SemaphoreType.DMA((3 * N_PIPE,)),
        ],
    )
    return pl.pallas_call(
        _moe_body,
        grid_spec=grid_spec,
        out_shape=jax.ShapeDtypeStruct((T_PER, D), jnp.float32),
        compiler_params=pltpu.CompilerParams(collective_id=1),
    )(fp, xf_bf, s2, w2, W1, W2)


def kernel(x, router, W1, W2):
    my_y = lax.axis_index("y")

    xfull, idxfull, wfull = _gate_exch(x, router.T)
    xf = xfull.reshape(2 * T_PER, D)
    idx_g = jnp.concatenate([idxfull[0].T, idxfull[1].T], axis=0)
    w_g = jnp.concatenate([wfull[0].T, wfull[1].T], axis=0)

    base = my_y * E_LOCAL
    ee = idx_g.reshape(-1)
    le = ee - base
    is_local = (le >= 0) & (le < E_LOCAL)
    lec = jnp.clip(le, 0, E_LOCAL - 1)
    onehot = (lec[:, None] == jnp.arange(E_LOCAL)[None, :]) & is_local[:, None]
    pos = jnp.cumsum(onehot.astype(jnp.int32), axis=0) - 1
    pos_a = jnp.sum(jnp.where(onehot, pos, 0), axis=1)
    valid = is_local & (pos_a < C)
    slot = jnp.where(valid, lec * C + pos_a, N_SLOTS)
    s2 = slot.reshape(2 * T_PER, K).T
    w2 = w_g.T.astype(jnp.float32)

    fp = (lax.axis_index("x") * 2 + lax.axis_index("z")).astype(jnp.int32)
    return _moe(fp[None], xf, s2, w2, W1, W2)


# device time: 140933 ns/iter; 1.1317x vs baseline; 1.1317x over previous
import jax
import jax.numpy as jnp
from jax import lax
from jax.experimental import pallas as pl
from jax.experimental.pallas import tpu as pltpu

T_PER = 1024
D = 1024
F = 4096
E_LOCAL = 8
K = 2
C = 320
N_SLOTS = E_LOCAL * C
FB = 512
NF = F // FB


def _peer():
    return (lax.axis_index("x"), 1 - lax.axis_index("y"), lax.axis_index("z"))


def _neighbor_barrier(peer):
    barrier_sem = pltpu.get_barrier_semaphore()
    pl.semaphore_signal(
        barrier_sem, inc=1, device_id=peer, device_id_type=pl.DeviceIdType.MESH
    )
    pl.semaphore_wait(barrier_sem, 1)


N_E = 2 * E_LOCAL


def _gate_exch_body(
    x_ref, rt_ref, xfull_ref, idxfull_ref, wfull_ref, rfull_ref,
    send_sems, recv_sems
):
    my_y = lax.axis_index("y")
    peer = _peer()
    _neighbor_barrier(peer)

    def exch(i, ref):
        return pltpu.make_async_remote_copy(
            src_ref=ref.at[my_y],
            dst_ref=ref.at[my_y],
            send_sem=send_sems.at[i],
            recv_sem=recv_sems.at[i],
            device_id=peer,
            device_id_type=pl.DeviceIdType.MESH,
        )

    xfull_ref[pl.ds(my_y, 1)] = x_ref[...].astype(jnp.bfloat16)[None]
    rd_x = exch(0, xfull_ref)
    rd_x.start()

    rfull_ref[pl.ds(my_y, 1)] = rt_ref[...][None]
    rd_r = exch(1, rfull_ref)
    rd_r.start()
    rd_r.wait()

    rcat = jnp.concatenate([rfull_ref[0], rfull_ref[1]], axis=0)
    g = lax.dot_general(
        x_ref[...], rcat, (((1,), (1,)), ((), ())),
        preferred_element_type=jnp.float32,
        precision=lax.Precision.HIGHEST,
    )
    iota16 = lax.broadcasted_iota(jnp.int32, (T_PER, N_E), 1)
    m1 = jnp.max(g, axis=1)
    i1 = jnp.argmax(g, axis=1).astype(jnp.int32)
    g2 = jnp.where(iota16 == i1[:, None], -jnp.inf, g)
    m2 = jnp.max(g2, axis=1)
    i2 = jnp.argmax(g2, axis=1).astype(jnp.int32)
    w1 = 1.0 / (1.0 + jnp.exp(m2 - m1))

    idxfull_ref[pl.ds(my_y, 1)] = jnp.concatenate(
        [i1[None, None, :], i2[None, None, :]], axis=1
    )
    wfull_ref[pl.ds(my_y, 1)] = jnp.concatenate(
        [w1[None, None, :], (1.0 - w1)[None, None, :]], axis=1
    )
    rd_i = exch(2, idxfull_ref)
    rd_w = exch(3, wfull_ref)
    rd_i.start()
    rd_w.start()
    rd_i.wait()
    rd_w.wait()
    rd_x.wait()


def _gate_exch(x, rt):
    return pl.pallas_call(
        _gate_exch_body,
        out_shape=(
            jax.ShapeDtypeStruct((2, T_PER, D), jnp.bfloat16),
            jax.ShapeDtypeStruct((2, K, T_PER), jnp.int32),
            jax.ShapeDtypeStruct((2, K, T_PER), jnp.float32),
        ),
        in_specs=[pl.BlockSpec(memory_space=pltpu.VMEM)] * 2,
        out_specs=(pl.BlockSpec(memory_space=pltpu.VMEM),) * 3,
        scratch_shapes=[
            pltpu.VMEM((2, E_LOCAL, D), jnp.float32),
            pltpu.SemaphoreType.DMA((4,)),
            pltpu.SemaphoreType.DMA((4,)),
        ],
        compiler_params=pltpu.CompilerParams(collective_id=0),
    )(x, rt)


A = 2 * T_PER * K

N_FP = 4
NF_PER = NF // N_FP


def _ffn_body(fp_ref, xf_ref, s2_ref, w2_ref_in, w1_ref, w2_ref, out_ref,
              xg_ref, acc_ref, wg_ref):
    e = pl.program_id(0)
    f = pl.program_id(1)

    @pl.when(f == 0)
    def _():
        iota_c = lax.broadcasted_iota(jnp.int32, (C, 2 * T_PER), 0) + e * C
        oh0 = iota_c == s2_ref[0][None, :]
        oh1 = iota_c == s2_ref[1][None, :]
        ohc = oh0.astype(jnp.bfloat16) + oh1.astype(jnp.bfloat16)
        xg_ref[...] = jnp.dot(
            ohc, xf_ref[...], preferred_element_type=jnp.float32
        ).astype(jnp.bfloat16)
        wg_ref[0, :] = jnp.sum(
            jnp.where(oh0, w2_ref_in[0][None, :], 0.0)
            + jnp.where(oh1, w2_ref_in[1][None, :], 0.0),
            axis=1,
        )
        acc_ref[...] = jnp.zeros_like(acc_ref)

    h = jnp.dot(
        xg_ref[...],
        w1_ref[0].astype(jnp.bfloat16),
        preferred_element_type=jnp.float32,
    )
    h = jnp.maximum(h, 0.0).astype(jnp.bfloat16)
    acc_ref[...] += jnp.dot(
        h, w2_ref[0].astype(jnp.bfloat16), preferred_element_type=jnp.float32
    )

    @pl.when(f == NF_PER - 1)
    def _():
        out_ref[0] = (acc_ref[...] * wg_ref[0, :][:, None]).astype(jnp.bfloat16)


def _ffn(fp, xf_bf, s2, w2, W1, W2):
    grid_spec = pltpu.PrefetchScalarGridSpec(
        num_scalar_prefetch=1,
        grid=(E_LOCAL, NF_PER),
        in_specs=[
            pl.BlockSpec((2 * T_PER, D), lambda e, f, fp: (0, 0)),
            pl.BlockSpec((K, 2 * T_PER), lambda e, f, fp: (0, 0)),
            pl.BlockSpec((K, 2 * T_PER), lambda e, f, fp: (0, 0)),
            pl.BlockSpec((1, D, FB), lambda e, f, fp: (e, 0, fp[0] * NF_PER + f)),
            pl.BlockSpec((1, FB, D), lambda e, f, fp: (e, fp[0] * NF_PER + f, 0)),
        ],
        out_specs=pl.BlockSpec((1, C, D), lambda e, f, fp: (e, 0, 0)),
        scratch_shapes=[
            pltpu.VMEM((C, D), jnp.bfloat16),
            pltpu.VMEM((C, D), jnp.float32),
            pltpu.VMEM((1, C), jnp.float32),
        ],
    )
    return pl.pallas_call(
        _ffn_body,
        grid_spec=grid_spec,
        out_shape=jax.ShapeDtypeStruct((E_LOCAL, C, D), jnp.bfloat16),
    )(fp, xf_bf, s2, w2, W1, W2)


N_PIPE = 8
SEG = T_PER // N_PIPE


def _undispatch_seg(s2_ref, y_ref, base):
    st0 = s2_ref[0, pl.ds(base, SEG)]
    st1 = s2_ref[1, pl.ds(base, SEG)]
    iota = lax.broadcasted_iota(jnp.int32, (SEG, N_SLOTS), 1)
    oh2 = (st0[:, None] == iota).astype(jnp.bfloat16) + (
        st1[:, None] == iota
    ).astype(jnp.bfloat16)
    return jnp.dot(oh2, y_ref[...], preferred_element_type=jnp.float32)


def _combine_body(
    y_ref, s2_ref, out_ref, acc_ref, send_y, send_x, send_z,
    rbuf_y, rbuf_x, rbuf_z, send_sems, recv_sems
):
    my_x = lax.axis_index("x")
    my_y = lax.axis_index("y")
    my_z = lax.axis_index("z")
    n_y = (my_x, 1 - my_y, my_z)
    n_x = (1 - my_x, my_y, my_z)
    n_z = (my_x, my_y, 1 - my_z)

    barrier_sem = pltpu.get_barrier_semaphore()
    for nbr in (n_y, n_x, n_z):
        pl.semaphore_signal(
            barrier_sem, inc=1, device_id=nbr,
            device_id_type=pl.DeviceIdType.MESH,
        )
    pl.semaphore_wait(barrier_sem, 3)

    def exchange(sem_i, src, dst, q, nbr):
        return pltpu.make_async_remote_copy(
            src_ref=src.at[pl.ds(q * SEG, SEG)],
            dst_ref=dst.at[pl.ds(q * SEG, SEG)],
            send_sem=send_sems.at[sem_i],
            recv_sem=recv_sems.at[sem_i],
            device_id=nbr,
            device_id_type=pl.DeviceIdType.MESH,
        )

    rd_y = [None] * N_PIPE
    rd_x = [None] * N_PIPE
    rd_z = [None] * N_PIPE

    for q in range(N_PIPE):
        send_y[pl.ds(q * SEG, SEG), :] = _undispatch_seg(
            s2_ref, y_ref, (1 - my_y) * T_PER + q * SEG
        ).astype(jnp.bfloat16)
        rd_y[q] = exchange(q, send_y, rbuf_y, q, n_y)
        rd_y[q].start()

    for q in range(N_PIPE):
        acc_ref[pl.ds(q * SEG, SEG), :] = _undispatch_seg(
            s2_ref, y_ref, my_y * T_PER + q * SEG
        )

    for q in range(N_PIPE):
        rd_y[q].wait()
        rows = pl.ds(q * SEG, SEG)
        acc_ref[rows, :] += rbuf_y[rows, :].astype(jnp.float32)
        send_x[rows, :] = acc_ref[rows, :].astype(jnp.bfloat16)
        rd_x[q] = exchange(N_PIPE + q, send_x, rbuf_x, q, n_x)
        rd_x[q].start()

    for q in range(N_PIPE):
        rd_x[q].wait()
        rows = pl.ds(q * SEG, SEG)
        acc_ref[rows, :] += rbuf_x[rows, :].astype(jnp.float32)
        send_z[rows, :] = acc_ref[rows, :].astype(jnp.bfloat16)
        rd_z[q] = exchange(2 * N_PIPE + q, send_z, rbuf_z, q, n_z)
        rd_z[q].start()

    for q in range(N_PIPE):
        rd_z[q].wait()
        rows = pl.ds(q * SEG, SEG)
        out_ref[rows, :] = acc_ref[rows, :] + rbuf_z[rows, :].astype(
            jnp.float32
        )


def _combine(yflat, s2):
    return pl.pallas_call(
        _combine_body,
        out_shape=jax.ShapeDtypeStruct((T_PER, D), jnp.float32),
        in_specs=[
            pl.BlockSpec(memory_space=pltpu.VMEM),
            pl.BlockSpec(memory_space=pltpu.VMEM),
        ],
        out_specs=pl.BlockSpec(memory_space=pltpu.VMEM),
        scratch_shapes=[
            pltpu.VMEM((T_PER, D), jnp.float32),
            pltpu.VMEM((T_PER, D), jnp.bfloat16),
            pltpu.VMEM((T_PER, D), jnp.bfloat16),
            pltpu.VMEM((T_PER, D), jnp.bfloat16),
            pltpu.VMEM((T_PER, D), jnp.bfloat16),
            pltpu.VMEM((T_PER, D), jnp.bfloat16),
            pltpu.VMEM((T_PER, D), jnp.bfloat16),
            pltpu.SemaphoreType.DMA((3 * N_PIPE,)),
            pltpu.SemaphoreType.DMA((3 * N_PIPE,)),
        ],
        compiler_params=pltpu.CompilerParams(collective_id=1),
    )(yflat, s2)


def kernel(x, router, W1, W2):
    my_y = lax.axis_index("y")

    xfull, idxfull, wfull = _gate_exch(x, router.T)
    xf = xfull.reshape(2 * T_PER, D)
    idx_g = jnp.concatenate([idxfull[0].T, idxfull[1].T], axis=0)
    w_g = jnp.concatenate([wfull[0].T, wfull[1].T], axis=0)

    base = my_y * E_LOCAL
    ee = idx_g.reshape(-1)
    ww = w_g.reshape(-1)
    le = ee - base
    is_local = (le >= 0) & (le < E_LOCAL)
    lec = jnp.clip(le, 0, E_LOCAL - 1)
    onehot = (lec[:, None] == jnp.arange(E_LOCAL)[None, :]) & is_local[:, None]
    pos = jnp.cumsum(onehot.astype(jnp.int32), axis=0) - 1
    pos_a = jnp.sum(jnp.where(onehot, pos, 0), axis=1)
    valid = is_local & (pos_a < C)
    slot = jnp.where(valid, lec * C + pos_a, N_SLOTS)
    s2 = slot.reshape(2 * T_PER, K).T
    w2 = w_g.T.astype(jnp.float32)

    fp = (lax.axis_index("x") * 2 + lax.axis_index("z")).astype(jnp.int32)
    yg = _ffn(fp[None], xf, s2, w2, W1, W2)

    return _combine(yg.reshape(N_SLOTS, D), s2)
